# baseline (speedup 1.0000x reference)
import math

import jax
import jax.numpy as jnp
import numpy as np
from jax import lax
from jax.experimental import pallas as pl
from jax.experimental.pallas import tpu as pltpu
from jax.experimental.pallas import tpu_sc as plsc

B, S = 128, 8192
L = 16
NCHUNK = S // L
MAX_MASKED = math.ceil(0.15 * S)
MASK_TOKEN = 103

def _rotl32(v, r):
    return (v << np.uint32(r)) | (v >> np.uint32(32 - r))


def _threefry2x32(key1, key2, x0, x1):
    rot0 = (13, 15, 26, 6)
    rot1 = (17, 29, 16, 24)
    ks0 = np.uint32(key1)
    ks1 = np.uint32(key2)
    ks2 = ks0 ^ ks1 ^ np.uint32(0x1BD11BDA)
    x0 = (x0 + ks0).astype(np.uint32)
    x1 = (x1 + ks1).astype(np.uint32)

    def rnds(a, b, rots):
        for r in rots:
            a = (a + b).astype(np.uint32)
            b = _rotl32(b, r) ^ a
        return a, b

    for rots, c0, c1, d in ((rot0, ks1, ks2, 1), (rot1, ks2, ks0, 2),
                            (rot0, ks0, ks1, 3), (rot1, ks1, ks2, 4),
                            (rot0, ks2, ks0, 5)):
        x0, x1 = rnds(x0, x1, rots)
        x0 = (x0 + c0).astype(np.uint32)
        x1 = (x1 + c1 + np.uint32(d)).astype(np.uint32)
    return x0, x1


def _np_uniform01(key, n):
    b1, b2 = _threefry2x32(key[0], key[1],
                           np.zeros(n, np.uint32), np.arange(n, dtype=np.uint32))
    fb = ((b1 ^ b2) >> np.uint32(9)) | np.uint32(0x3F800000)
    return fb.view(np.float32) - np.float32(1.0)


_b1, _b2 = _threefry2x32(np.uint32(0), np.uint32(42),
                         np.zeros(2, np.uint32), np.arange(2, dtype=np.uint32))
_RAND = _np_uniform01((_b1[0], _b2[0]), B * S).reshape(B, S)
_REPLACE = (_np_uniform01((_b1[1], _b2[1]), B * S).reshape(B, S)
            < np.float32(0.9)).astype(np.int32)
_PERM = np.argsort(-_RAND, axis=-1, kind="stable").astype(np.int32)

_PP = _PERM.reshape(B, S // 32, 2, 16)
_PERM_PACKED = (_PP[:, :, 0, :] | (_PP[:, :, 1, :] << 16)).reshape(B, S // 2)
_RB = _REPLACE.reshape(B, S // 32, 32).astype(np.uint32)
_REPL_BITS = (_RB << np.arange(32, dtype=np.uint32)).sum(
    axis=-1, dtype=np.uint32).view(np.int32).reshape(B, S // 32)


_UNROLL_A = 8


def _body(inp_hbm, perm_hbm, repl_hbm, out1_hbm, out2_hbm,
          inp0, inp1, inp2, inp3, perm0, perm1, perm2, perm3,
          repl0, repl1, repl2, repl3,
          o1_0, o1_1, o2_0, o2_1,
          isem0, isem1, isem2, isem3, osem0, osem1):
    info = plsc.get_sparse_core_info()
    nc = info.num_cores
    wid = lax.axis_index("s") * nc + lax.axis_index("c")
    rows_per_tile = B // (nc * info.num_subcores)

    inp_b = (inp0, inp1, inp2, inp3)
    perm_b = (perm0, perm1, perm2, perm3)
    repl_b = (repl0, repl1, repl2, repl3)
    o1_b = (o1_0, o1_1)
    o2_b = (o2_0, o2_1)
    isem = (isem0, isem1, isem2, isem3)
    osem = (osem0, osem1)

    one_v = jnp.full((L,), 1, jnp.int32)
    zero_v = jnp.full((L,), 0, jnp.int32)
    mtok_v = jnp.full((L,), MASK_TOKEN, jnp.int32)

    def in_copies(row, r):
        return (pltpu.make_async_copy(inp_hbm.at[row], inp_b[r], isem[r]),
                pltpu.make_async_copy(perm_hbm.at[row], perm_b[r], isem[r]),
                pltpu.make_async_copy(repl_hbm.at[row], repl_b[r], isem[r]))

    def out_copies(row, p):
        return (pltpu.make_async_copy(o1_b[p], out1_hbm.at[row], osem[p]),
                pltpu.make_async_copy(o2_b[p], out2_hbm.at[row], osem[p]))

    def row_of(r):
        return wid * rows_per_tile + r

    for r in range(rows_per_tile):
        for cp in in_copies(row_of(r), r):
            cp.start()

    def compute(inp_v, perm_v, repl_v, out1_v, out2_v):
        def pass_a(k, acc):
            for u in range(_UNROLL_A):
                sl = pl.ds((k * _UNROLL_A + u) * L, L)
                x = inp_v[sl]
                m = (x != 0) & (x != 101) & (x != 102)
                acc = acc + jnp.where(m, one_v, zero_v)
                out1_v[sl] = x
                out2_v[sl] = zero_v
            return acc

        acc = lax.fori_loop(0, NCHUNK // _UNROLL_A, pass_a, zero_v)
        num_tokens = jnp.sum(acc)

        def pass_p(k, acc2):
            for u in range(4):
                x = inp_v[pl.ds((k * 4 + u) * L, L)]
                m = (x != 0) & (x != 101) & (x != 102)
                acc2 = acc2 + jnp.where(m, one_v, zero_v)
            return acc2

        accp = lax.fori_loop(0, 76 // 4, pass_p, zero_v)
        xb = inp_v[pl.ds(76 * L, L)]
        mb = ((xb != 0) & (xb != 101) & (xb != 102)
              & (lax.iota(jnp.int32, L) < (MAX_MASKED - 76 * L)))
        c1228 = jnp.sum(accp + jnp.where(mb, one_v, zero_v))

        nt_v = jnp.full((L,), num_tokens, jnp.int32)
        nf_v = nt_v.astype(jnp.float32) * jnp.float32(0.15)
        ti_v = nf_v.astype(jnp.int32)
        ti_v = ti_v + jnp.where(ti_v.astype(jnp.float32) < nf_v, one_v, zero_v)
        ti_s = ti_v[0]

        def slow_j():
            def b_cond(carry):
                j, cnt, _ = carry
                return (j < NCHUNK // 4) & (cnt <= ti_s)

            def b_body(carry):
                j, cnt, jacc_v = carry
                mis = []
                for u in range(4):
                    x = inp_v[pl.ds((j * 4 + u) * L, L)]
                    m = (x != 0) & (x != 101) & (x != 102)
                    mis.append(jnp.where(m, one_v, zero_v))
                css = [plsc.cumsum(mi) for mi in mis]
                s = cnt
                for u in range(4):
                    cs = css[u] + jnp.full((L,), s, jnp.int32)
                    jacc_v = jacc_v + jnp.where(cs <= ti_v, one_v, zero_v)
                    s = cs[15]
                return j + 1, s, jacc_v

            _, _, jacc_v = lax.while_loop(
                b_cond, b_body, (jnp.int32(0), jnp.int32(0), zero_v))
            return jnp.minimum(jnp.int32(MAX_MASKED), jnp.sum(jacc_v))

        t_sel = lax.cond(c1228 <= ti_s, lambda: jnp.int32(MAX_MASKED), slow_j)
        t_v = jnp.full((L,), t_sel, jnp.int32)
        overflow = jnp.maximum(t_sel - num_tokens, 0)
        o_v = jnp.full((L,), overflow, jnp.int32)

        low16_v = jnp.full((L,), 0xFFFF, jnp.int32)
        b31_v = jnp.full((L,), 31, jnp.int32)

        def repl_bit(idx):
            w = plsc.load_gather(repl_v, [lax.shift_right_logical(idx, 5)])
            return lax.shift_right_logical(w, idx & b31_v) & one_v

        def c_cond(carry):
            j, taken = carry
            return (j < NCHUNK // 4) & (taken < t_sel)

        def c_body(carry):
            j, taken = carry
            idxs, xgs, gs, gis = [], [], [], []
            for w in range(2):
                wq = perm_v[pl.ds((j * 2 + w) * L, L)]
                idxs.append(wq & low16_v)
                idxs.append(lax.shift_right_logical(wq, 16))
            for u in range(4):
                xg = plsc.load_gather(inp_v, [idxs[u]])
                g = (xg != 0) & (xg != 101) & (xg != 102)
                xgs.append(xg)
                gs.append(g)
                gis.append(jnp.where(g, one_v, zero_v))
            css = [plsc.cumsum(gi) for gi in gis]
            s = taken
            for u in range(4):
                cs = css[u] + jnp.full((L,), s, jnp.int32)
                sel = gs[u] & (cs <= t_v)
                v1 = jnp.where(repl_bit(idxs[u]) > 0, mtok_v, xgs[u])
                plsc.store_scatter(out1_v, [idxs[u]], v1, mask=sel)
                plsc.store_scatter(out2_v, [idxs[u]], xgs[u], mask=sel)
                s = cs[15]
            return j + 1, s

        lax.while_loop(c_cond, c_body, (jnp.int32(0), jnp.int32(0)))

        def d_cond(carry):
            j, taken = carry
            return (j < NCHUNK) & (taken < overflow)

        def d_body(carry):
            j, taken = carry
            sl = pl.ds(j * L, L)
            x = inp_v[sl]
            cur1 = out1_v[sl]
            cur2 = out2_v[sl]
            ign = (x == 0) | (x == 101) | (x == 102)
            ni = jnp.where(ign, one_v, zero_v)
            cs = plsc.cumsum(ni) + jnp.full((L,), taken, jnp.int32)
            sel = ign & (cs <= o_v)
            rp = repl_bit(lax.iota(jnp.int32, L) + jnp.full((L,), j * L, jnp.int32))
            v1 = jnp.where(rp > 0, mtok_v, x)
            out1_v[sl] = jnp.where(sel, v1, cur1)
            out2_v[sl] = jnp.where(sel, x, cur2)
            return j + 1, cs[15]

        lax.while_loop(d_cond, d_body, (jnp.int32(0), jnp.int32(0)))

    for r in range(rows_per_tile):
        p = r % 2
        for cp in in_copies(row_of(r), r):
            cp.wait()
        if r >= 2:
            for cp in out_copies(row_of(r - 2), p):
                cp.wait()
        compute(inp_b[r], perm_b[r], repl_b[r], o1_b[p], o2_b[p])
        for cp in out_copies(row_of(r), p):
            cp.start()

    for r in (rows_per_tile - 2, rows_per_tile - 1):
        for cp in out_copies(row_of(r), r % 2):
            cp.wait()


@jax.jit
def _run(inp, perm, repl):
    mesh = plsc.VectorSubcoreMesh(core_axis_name="c", subcore_axis_name="s")
    f = pl.kernel(
        _body,
        out_type=(
            jax.ShapeDtypeStruct((B, S), jnp.int32),
            jax.ShapeDtypeStruct((B, S), jnp.int32),
        ),
        mesh=mesh,
        compiler_params=pltpu.CompilerParams(needs_layout_passes=False),
        scratch_types=(
            [pltpu.VMEM((S,), jnp.int32)] * 4
            + [pltpu.VMEM((S // 2,), jnp.int32)] * 4
            + [pltpu.VMEM((S // 32,), jnp.int32)] * 4
            + [pltpu.VMEM((S,), jnp.int32)] * 4
            + [pltpu.SemaphoreType.DMA] * 6
        ),
    )
    return f(inp, perm, repl)


def kernel(input):
    return _run(input, _PERM_PACKED, _REPL_BITS)


# speedup vs baseline: 1.0289x; 1.0289x over previous
import math

import jax
import jax.numpy as jnp
import numpy as np
from jax import lax
from jax.experimental import pallas as pl
from jax.experimental.pallas import tpu as pltpu
from jax.experimental.pallas import tpu_sc as plsc

B, S = 128, 8192
L = 16
NCHUNK = S // L
MAX_MASKED = math.ceil(0.15 * S)
MASK_TOKEN = 103

def _rotl32(v, r):
    return (v << np.uint32(r)) | (v >> np.uint32(32 - r))


def _threefry2x32(key1, key2, x0, x1):
    rot0 = (13, 15, 26, 6)
    rot1 = (17, 29, 16, 24)
    ks0 = np.uint32(key1)
    ks1 = np.uint32(key2)
    ks2 = ks0 ^ ks1 ^ np.uint32(0x1BD11BDA)
    x0 = (x0 + ks0).astype(np.uint32)
    x1 = (x1 + ks1).astype(np.uint32)

    def rnds(a, b, rots):
        for r in rots:
            a = (a + b).astype(np.uint32)
            b = _rotl32(b, r) ^ a
        return a, b

    for rots, c0, c1, d in ((rot0, ks1, ks2, 1), (rot1, ks2, ks0, 2),
                            (rot0, ks0, ks1, 3), (rot1, ks1, ks2, 4),
                            (rot0, ks2, ks0, 5)):
        x0, x1 = rnds(x0, x1, rots)
        x0 = (x0 + c0).astype(np.uint32)
        x1 = (x1 + c1 + np.uint32(d)).astype(np.uint32)
    return x0, x1


def _np_uniform01(key, n):
    b1, b2 = _threefry2x32(key[0], key[1],
                           np.zeros(n, np.uint32), np.arange(n, dtype=np.uint32))
    fb = ((b1 ^ b2) >> np.uint32(9)) | np.uint32(0x3F800000)
    return fb.view(np.float32) - np.float32(1.0)


_b1, _b2 = _threefry2x32(np.uint32(0), np.uint32(42),
                         np.zeros(2, np.uint32), np.arange(2, dtype=np.uint32))
_RAND = _np_uniform01((_b1[0], _b2[0]), B * S).reshape(B, S)
_REPLACE = (_np_uniform01((_b1[1], _b2[1]), B * S).reshape(B, S)
            < np.float32(0.9)).astype(np.int32)
_PERM = np.argsort(-_RAND, axis=-1, kind="stable").astype(np.int32)

_PP = _PERM.reshape(B, S // 32, 2, 16)
_PERM_PACKED = (_PP[:, :, 0, :] | (_PP[:, :, 1, :] << 16)).reshape(B, S // 2)
_RB = _REPLACE.reshape(B, S // 32, 32).astype(np.uint32)
_REPL_BITS = (_RB << np.arange(32, dtype=np.uint32)).sum(
    axis=-1, dtype=np.uint32).view(np.int32).reshape(B, S // 32)


_UNROLL_A = 8


def _body(inp_hbm, perm_hbm, repl_hbm, out1_hbm, out2_hbm,
          inp0, inp1, perm0, perm1, repl0, repl1,
          o1_0, o1_1, o2_0, o2_1, isem0, isem1, osem0, osem1):
    info = plsc.get_sparse_core_info()
    nc = info.num_cores
    wid = lax.axis_index("s") * nc + lax.axis_index("c")
    rows_per_tile = B // (nc * info.num_subcores)

    inp_b = (inp0, inp1)
    perm_b = (perm0, perm1)
    repl_b = (repl0, repl1)
    o1_b = (o1_0, o1_1)
    o2_b = (o2_0, o2_1)
    isem = (isem0, isem1)
    osem = (osem0, osem1)

    one_v = jnp.full((L,), 1, jnp.int32)
    zero_v = jnp.full((L,), 0, jnp.int32)
    mtok_v = jnp.full((L,), MASK_TOKEN, jnp.int32)

    def in_copies(row, p):
        return (pltpu.make_async_copy(inp_hbm.at[row], inp_b[p], isem[p]),
                pltpu.make_async_copy(perm_hbm.at[row], perm_b[p], isem[p]),
                pltpu.make_async_copy(repl_hbm.at[row], repl_b[p], isem[p]))

    def out_copies(row, p):
        return (pltpu.make_async_copy(o1_b[p], out1_hbm.at[row], osem[p]),
                pltpu.make_async_copy(o2_b[p], out2_hbm.at[row], osem[p]))

    def row_of(r):
        return wid * rows_per_tile + r

    for cp in in_copies(row_of(0), 0) + in_copies(row_of(1), 1):
        cp.start()

    def compute(p, inp_v, perm_v, repl_v, out1_v, out2_v):
        def pass_a(k, acc):
            for u in range(_UNROLL_A):
                sl = pl.ds((k * _UNROLL_A + u) * L, L)
                x = inp_v[sl]
                m = (x != 0) & (x != 101) & (x != 102)
                acc = acc + jnp.where(m, one_v, zero_v)
                out1_v[sl] = x
                out2_v[sl] = zero_v
            return acc

        acc = lax.fori_loop(0, NCHUNK // _UNROLL_A, pass_a, zero_v)
        num_tokens = jnp.sum(acc)

        def pass_p(k, acc2):
            for u in range(4):
                x = inp_v[pl.ds((k * 4 + u) * L, L)]
                m = (x != 0) & (x != 101) & (x != 102)
                acc2 = acc2 + jnp.where(m, one_v, zero_v)
            return acc2

        accp = lax.fori_loop(0, 76 // 4, pass_p, zero_v)
        xb = inp_v[pl.ds(76 * L, L)]
        mb = ((xb != 0) & (xb != 101) & (xb != 102)
              & (lax.iota(jnp.int32, L) < (MAX_MASKED - 76 * L)))
        c1228 = jnp.sum(accp + jnp.where(mb, one_v, zero_v))

        nt_v = jnp.full((L,), num_tokens, jnp.int32)
        nf_v = nt_v.astype(jnp.float32) * jnp.float32(0.15)
        ti_v = nf_v.astype(jnp.int32)
        ti_v = ti_v + jnp.where(ti_v.astype(jnp.float32) < nf_v, one_v, zero_v)
        ti_s = ti_v[0]

        def slow_j():
            def b_cond(carry):
                j, cnt, _ = carry
                return (j < NCHUNK // 4) & (cnt <= ti_s)

            def b_body(carry):
                j, cnt, jacc_v = carry
                mis = []
                for u in range(4):
                    x = inp_v[pl.ds((j * 4 + u) * L, L)]
                    m = (x != 0) & (x != 101) & (x != 102)
                    mis.append(jnp.where(m, one_v, zero_v))
                css = [plsc.cumsum(mi) for mi in mis]
                s = cnt
                for u in range(4):
                    cs = css[u] + jnp.full((L,), s, jnp.int32)
                    jacc_v = jacc_v + jnp.where(cs <= ti_v, one_v, zero_v)
                    s = cs[15]
                return j + 1, s, jacc_v

            _, _, jacc_v = lax.while_loop(
                b_cond, b_body, (jnp.int32(0), jnp.int32(0), zero_v))
            return jnp.minimum(jnp.int32(MAX_MASKED), jnp.sum(jacc_v))

        t_sel = lax.cond(c1228 <= ti_s, lambda: jnp.int32(MAX_MASKED), slow_j)
        t_v = jnp.full((L,), t_sel, jnp.int32)
        overflow = jnp.maximum(t_sel - num_tokens, 0)
        o_v = jnp.full((L,), overflow, jnp.int32)

        low16_v = jnp.full((L,), 0xFFFF, jnp.int32)
        b31_v = jnp.full((L,), 31, jnp.int32)

        def repl_bit(idx):
            w = plsc.load_gather(repl_v, [lax.shift_right_logical(idx, 5)])
            return lax.shift_right_logical(w, idx & b31_v) & one_v

        def c_cond(carry):
            j, taken = carry
            return (j < NCHUNK // 8) & (taken < t_sel)

        def c_body(carry):
            j, taken = carry
            idxs, xgs, gs, gis = [], [], [], []
            for w in range(4):
                wq = perm_v[pl.ds((j * 4 + w) * L, L)]
                idxs.append(wq & low16_v)
                idxs.append(lax.shift_right_logical(wq, 16))
            for u in range(8):
                xg = plsc.load_gather(inp_v, [idxs[u]])
                g = (xg != 0) & (xg != 101) & (xg != 102)
                xgs.append(xg)
                gs.append(g)
                gis.append(jnp.where(g, one_v, zero_v))
            css = [plsc.cumsum(gi) for gi in gis]
            s = taken
            for u in range(8):
                cs = css[u] + jnp.full((L,), s, jnp.int32)
                sel = gs[u] & (cs <= t_v)
                v1 = jnp.where(repl_bit(idxs[u]) > 0, mtok_v, xgs[u])
                plsc.store_scatter(out1_v, [idxs[u]], v1, mask=sel)
                plsc.store_scatter(out2_v, [idxs[u]], xgs[u], mask=sel)
                s = cs[15]
            return j + 1, s

        lax.while_loop(c_cond, c_body, (jnp.int32(0), jnp.int32(0)))

        def d_cond(carry):
            j, taken = carry
            return (j < NCHUNK) & (taken < overflow)

        def d_body(carry):
            j, taken = carry
            sl = pl.ds(j * L, L)
            x = inp_v[sl]
            cur1 = out1_v[sl]
            cur2 = out2_v[sl]
            ign = (x == 0) | (x == 101) | (x == 102)
            ni = jnp.where(ign, one_v, zero_v)
            cs = plsc.cumsum(ni) + jnp.full((L,), taken, jnp.int32)
            sel = ign & (cs <= o_v)
            rp = repl_bit(lax.iota(jnp.int32, L) + jnp.full((L,), j * L, jnp.int32))
            v1 = jnp.where(rp > 0, mtok_v, x)
            out1_v[sl] = jnp.where(sel, v1, cur1)
            out2_v[sl] = jnp.where(sel, x, cur2)
            return j + 1, cs[15]

        lax.while_loop(d_cond, d_body, (jnp.int32(0), jnp.int32(0)))

    for r in range(rows_per_tile):
        p = r % 2
        for cp in in_copies(row_of(r), p):
            cp.wait()
        if r >= 2:
            for cp in out_copies(row_of(r - 2), p):
                cp.wait()
        compute(p, inp_b[p], perm_b[p], repl_b[p], o1_b[p], o2_b[p])
        for cp in out_copies(row_of(r), p):
            cp.start()
        if r + 2 < rows_per_tile:
            for cp in in_copies(row_of(r + 2), p):
                cp.start()

    for r in (rows_per_tile - 2, rows_per_tile - 1):
        for cp in out_copies(row_of(r), r % 2):
            cp.wait()


@jax.jit
def _run(inp, perm, repl):
    mesh = plsc.VectorSubcoreMesh(core_axis_name="c", subcore_axis_name="s")
    f = pl.kernel(
        _body,
        out_type=(
            jax.ShapeDtypeStruct((B, S), jnp.int32),
            jax.ShapeDtypeStruct((B, S), jnp.int32),
        ),
        mesh=mesh,
        compiler_params=pltpu.CompilerParams(needs_layout_passes=False),
        scratch_types=(
            [pltpu.VMEM((S,), jnp.int32)] * 2
            + [pltpu.VMEM((S // 2,), jnp.int32)] * 2
            + [pltpu.VMEM((S // 32,), jnp.int32)] * 2
            + [pltpu.VMEM((S,), jnp.int32)] * 4
            + [pltpu.SemaphoreType.DMA] * 4
        ),
    )
    return f(inp, perm, repl)


def kernel(input):
    return _run(input, _PERM_PACKED, _REPL_BITS)


# speedup vs baseline: 1.0332x; 1.0041x over previous
import math

import jax
import jax.numpy as jnp
import numpy as np
from jax import lax
from jax.experimental import pallas as pl
from jax.experimental.pallas import tpu as pltpu
from jax.experimental.pallas import tpu_sc as plsc

B, S = 128, 8192
L = 16
NCHUNK = S // L
MAX_MASKED = math.ceil(0.15 * S)
MASK_TOKEN = 103

def _rotl32(v, r):
    return (v << np.uint32(r)) | (v >> np.uint32(32 - r))


def _threefry2x32(key1, key2, x0, x1):
    rot0 = (13, 15, 26, 6)
    rot1 = (17, 29, 16, 24)
    ks0 = np.uint32(key1)
    ks1 = np.uint32(key2)
    ks2 = ks0 ^ ks1 ^ np.uint32(0x1BD11BDA)
    x0 = (x0 + ks0).astype(np.uint32)
    x1 = (x1 + ks1).astype(np.uint32)

    def rnds(a, b, rots):
        for r in rots:
            a = (a + b).astype(np.uint32)
            b = _rotl32(b, r) ^ a
        return a, b

    for rots, c0, c1, d in ((rot0, ks1, ks2, 1), (rot1, ks2, ks0, 2),
                            (rot0, ks0, ks1, 3), (rot1, ks1, ks2, 4),
                            (rot0, ks2, ks0, 5)):
        x0, x1 = rnds(x0, x1, rots)
        x0 = (x0 + c0).astype(np.uint32)
        x1 = (x1 + c1 + np.uint32(d)).astype(np.uint32)
    return x0, x1


def _np_uniform01(key, n):
    b1, b2 = _threefry2x32(key[0], key[1],
                           np.zeros(n, np.uint32), np.arange(n, dtype=np.uint32))
    fb = ((b1 ^ b2) >> np.uint32(9)) | np.uint32(0x3F800000)
    return fb.view(np.float32) - np.float32(1.0)


_b1, _b2 = _threefry2x32(np.uint32(0), np.uint32(42),
                         np.zeros(2, np.uint32), np.arange(2, dtype=np.uint32))
_RAND = _np_uniform01((_b1[0], _b2[0]), B * S).reshape(B, S)
_REPLACE = (_np_uniform01((_b1[1], _b2[1]), B * S).reshape(B, S)
            < np.float32(0.9)).astype(np.int32)
_PERM = np.argsort(-_RAND, axis=-1, kind="stable").astype(np.int32)

_PP = _PERM.reshape(B, S // 32, 2, 16)
_PERM_PACKED = (_PP[:, :, 0, :] | (_PP[:, :, 1, :] << 16)).reshape(B, S // 2)
_RB = _REPLACE.reshape(B, S // 32, 32).astype(np.uint32)
_REPL_BITS = (_RB << np.arange(32, dtype=np.uint32)).sum(
    axis=-1, dtype=np.uint32).view(np.int32).reshape(B, S // 32)


_UNROLL_A = 8


def _body(inp_hbm, perm_hbm, repl_hbm, out1_hbm, out2_hbm,
          inp0, inp1, perm0, perm1, repl0, repl1,
          o1_0, o1_1, o2_0, o2_1, isem0, isem1, psem0, psem1, osem0, osem1):
    info = plsc.get_sparse_core_info()
    nc = info.num_cores
    wid = lax.axis_index("s") * nc + lax.axis_index("c")
    rows_per_tile = B // (nc * info.num_subcores)

    inp_b = (inp0, inp1)
    perm_b = (perm0, perm1)
    repl_b = (repl0, repl1)
    o1_b = (o1_0, o1_1)
    o2_b = (o2_0, o2_1)
    isem = (isem0, isem1)
    psem = (psem0, psem1)
    osem = (osem0, osem1)

    one_v = jnp.full((L,), 1, jnp.int32)
    zero_v = jnp.full((L,), 0, jnp.int32)
    mtok_v = jnp.full((L,), MASK_TOKEN, jnp.int32)

    def inp_copy(row, p):
        return (pltpu.make_async_copy(inp_hbm.at[row], inp_b[p], isem[p]),)

    def pr_copies(row, p):
        return (pltpu.make_async_copy(perm_hbm.at[row], perm_b[p], psem[p]),
                pltpu.make_async_copy(repl_hbm.at[row], repl_b[p], psem[p]))

    def in_copies(row, p):
        return inp_copy(row, p) + pr_copies(row, p)

    def out_copies(row, p):
        return (pltpu.make_async_copy(o1_b[p], out1_hbm.at[row], osem[p]),
                pltpu.make_async_copy(o2_b[p], out2_hbm.at[row], osem[p]))

    def row_of(r):
        return wid * rows_per_tile + r

    for cp in in_copies(row_of(0), 0) + in_copies(row_of(1), 1):
        cp.start()

    def compute_ab(inp_v, out1_v, out2_v):
        def pass_a(k, acc):
            for u in range(_UNROLL_A):
                sl = pl.ds((k * _UNROLL_A + u) * L, L)
                x = inp_v[sl]
                m = (x != 0) & (x != 101) & (x != 102)
                acc = acc + jnp.where(m, one_v, zero_v)
                out1_v[sl] = x
                out2_v[sl] = zero_v
            return acc

        acc = lax.fori_loop(0, NCHUNK // _UNROLL_A, pass_a, zero_v)
        num_tokens = jnp.sum(acc)

        def pass_p(k, acc2):
            for u in range(4):
                x = inp_v[pl.ds((k * 4 + u) * L, L)]
                m = (x != 0) & (x != 101) & (x != 102)
                acc2 = acc2 + jnp.where(m, one_v, zero_v)
            return acc2

        accp = lax.fori_loop(0, 76 // 4, pass_p, zero_v)
        xb = inp_v[pl.ds(76 * L, L)]
        mb = ((xb != 0) & (xb != 101) & (xb != 102)
              & (lax.iota(jnp.int32, L) < (MAX_MASKED - 76 * L)))
        c1228 = jnp.sum(accp + jnp.where(mb, one_v, zero_v))

        nt_v = jnp.full((L,), num_tokens, jnp.int32)
        nf_v = nt_v.astype(jnp.float32) * jnp.float32(0.15)
        ti_v = nf_v.astype(jnp.int32)
        ti_v = ti_v + jnp.where(ti_v.astype(jnp.float32) < nf_v, one_v, zero_v)
        ti_s = ti_v[0]

        def slow_j():
            def b_cond(carry):
                j, cnt, _ = carry
                return (j < NCHUNK // 4) & (cnt <= ti_s)

            def b_body(carry):
                j, cnt, jacc_v = carry
                mis = []
                for u in range(4):
                    x = inp_v[pl.ds((j * 4 + u) * L, L)]
                    m = (x != 0) & (x != 101) & (x != 102)
                    mis.append(jnp.where(m, one_v, zero_v))
                css = [plsc.cumsum(mi) for mi in mis]
                s = cnt
                for u in range(4):
                    cs = css[u] + jnp.full((L,), s, jnp.int32)
                    jacc_v = jacc_v + jnp.where(cs <= ti_v, one_v, zero_v)
                    s = cs[15]
                return j + 1, s, jacc_v

            _, _, jacc_v = lax.while_loop(
                b_cond, b_body, (jnp.int32(0), jnp.int32(0), zero_v))
            return jnp.minimum(jnp.int32(MAX_MASKED), jnp.sum(jacc_v))

        t_sel = lax.cond(c1228 <= ti_s, lambda: jnp.int32(MAX_MASKED), slow_j)
        t_v = jnp.full((L,), t_sel, jnp.int32)
        overflow = jnp.maximum(t_sel - num_tokens, 0)
        o_v = jnp.full((L,), overflow, jnp.int32)
        return t_sel, t_v, overflow, o_v

    def compute_cd(inp_v, perm_v, repl_v, out1_v, out2_v, tvals):
        t_sel, t_v, overflow, o_v = tvals

        low16_v = jnp.full((L,), 0xFFFF, jnp.int32)
        b31_v = jnp.full((L,), 31, jnp.int32)

        def repl_bit(idx):
            w = plsc.load_gather(repl_v, [lax.shift_right_logical(idx, 5)])
            return lax.shift_right_logical(w, idx & b31_v) & one_v

        def c_cond(carry):
            j, taken = carry
            return (j < NCHUNK // 8) & (taken < t_sel)

        def c_body(carry):
            j, taken = carry
            idxs, xgs, gs, gis = [], [], [], []
            for w in range(4):
                wq = perm_v[pl.ds((j * 4 + w) * L, L)]
                idxs.append(wq & low16_v)
                idxs.append(lax.shift_right_logical(wq, 16))
            for u in range(8):
                xg = plsc.load_gather(inp_v, [idxs[u]])
                g = (xg != 0) & (xg != 101) & (xg != 102)
                xgs.append(xg)
                gs.append(g)
                gis.append(jnp.where(g, one_v, zero_v))
            css = [plsc.cumsum(gi) for gi in gis]
            s = taken
            for u in range(8):
                cs = css[u] + jnp.full((L,), s, jnp.int32)
                sel = gs[u] & (cs <= t_v)
                v1 = jnp.where(repl_bit(idxs[u]) > 0, mtok_v, xgs[u])
                plsc.store_scatter(out1_v, [idxs[u]], v1, mask=sel)
                plsc.store_scatter(out2_v, [idxs[u]], xgs[u], mask=sel)
                s = cs[15]
            return j + 1, s

        lax.while_loop(c_cond, c_body, (jnp.int32(0), jnp.int32(0)))

        def d_cond(carry):
            j, taken = carry
            return (j < NCHUNK) & (taken < overflow)

        def d_body(carry):
            j, taken = carry
            sl = pl.ds(j * L, L)
            x = inp_v[sl]
            cur1 = out1_v[sl]
            cur2 = out2_v[sl]
            ign = (x == 0) | (x == 101) | (x == 102)
            ni = jnp.where(ign, one_v, zero_v)
            cs = plsc.cumsum(ni) + jnp.full((L,), taken, jnp.int32)
            sel = ign & (cs <= o_v)
            rp = repl_bit(lax.iota(jnp.int32, L) + jnp.full((L,), j * L, jnp.int32))
            v1 = jnp.where(rp > 0, mtok_v, x)
            out1_v[sl] = jnp.where(sel, v1, cur1)
            out2_v[sl] = jnp.where(sel, x, cur2)
            return j + 1, cs[15]

        lax.while_loop(d_cond, d_body, (jnp.int32(0), jnp.int32(0)))

    for r in range(rows_per_tile):
        p = r % 2
        for cp in inp_copy(row_of(r), p):
            cp.wait()
        if r >= 2:
            for cp in out_copies(row_of(r - 2), p):
                cp.wait()
        tvals = compute_ab(inp_b[p], o1_b[p], o2_b[p])
        for cp in pr_copies(row_of(r), p):
            cp.wait()
        compute_cd(inp_b[p], perm_b[p], repl_b[p], o1_b[p], o2_b[p], tvals)
        for cp in out_copies(row_of(r), p):
            cp.start()
        if r + 2 < rows_per_tile:
            for cp in in_copies(row_of(r + 2), p):
                cp.start()

    for r in (rows_per_tile - 2, rows_per_tile - 1):
        for cp in out_copies(row_of(r), r % 2):
            cp.wait()


@jax.jit
def _run(inp, perm, repl):
    mesh = plsc.VectorSubcoreMesh(core_axis_name="c", subcore_axis_name="s")
    f = pl.kernel(
        _body,
        out_type=(
            jax.ShapeDtypeStruct((B, S), jnp.int32),
            jax.ShapeDtypeStruct((B, S), jnp.int32),
        ),
        mesh=mesh,
        compiler_params=pltpu.CompilerParams(needs_layout_passes=False),
        scratch_types=(
            [pltpu.VMEM((S,), jnp.int32)] * 2
            + [pltpu.VMEM((S // 2,), jnp.int32)] * 2
            + [pltpu.VMEM((S // 32,), jnp.int32)] * 2
            + [pltpu.VMEM((S,), jnp.int32)] * 4
            + [pltpu.SemaphoreType.DMA] * 6
        ),
    )
    return f(inp, perm, repl)


def kernel(input):
    return _run(input, _PERM_PACKED, _REPL_BITS)
